# baseline (speedup 1.0000x reference)
import functools

import jax
import jax.numpy as jnp
from jax import lax
from jax.experimental import pallas as pl
from jax.experimental.pallas import tpu as pltpu
from jax.experimental.pallas import tpu_sc as plsc

N = 10000
E = 160000
F = 256
H = 512
C = 40
W = 64

NC = 2
NS = 16
CH = 128
RING = 4
EROWS = 1280
E2 = EROWS * CH
SINK = N
NP = N + 16
RPT = EROWS // NS
HPT = RPT // 2
NG = RPT // RING
RCH = 80
NRC = N // RCH
KMAX = (NRC + NS - 1) // NS


def _mask_body(k, w_ref, s_ref, o_ref):
    s = s_ref[...]
    bits = lax.bitcast_convert_type(s, jnp.int32)

    def step(_, lohi):
        lo, hi = lohi
        mid = (lo + hi) // 2
        cnt = jnp.sum((bits <= mid).astype(jnp.int32))
        take = cnt >= k + 1
        return jnp.where(take, lo, mid + 1), jnp.where(take, mid, hi)

    lo, _ = lax.fori_loop(0, 31, step, (jnp.int32(0), jnp.int32(0x3F800000)))
    thr = lax.bitcast_convert_type(lo, jnp.float32)
    o_ref[...] = jnp.where(s < thr, 0.0, w_ref[...]).astype(jnp.bfloat16)


def _mask_weights(w, s, k):
    return pl.pallas_call(
        functools.partial(_mask_body, k),
        out_shape=jax.ShapeDtypeStruct(w.shape, jnp.bfloat16),
    )(w, s)


def _make_sc_agg(ntab, with_deg, edge_split):
    mesh = plsc.VectorSubcoreMesh(
        core_axis_name="c", subcore_axis_name="s", num_cores=NC,
        num_subcores=NS)
    nout = ntab * (NC if edge_split else 1)
    n_out_total = nout + (2 if with_deg else 0)
    rpt = RPT if not edge_split else RPT // NC

    def body(*refs):
        tabs = refs[:ntab]
        src_ref, dst_ref, zrows = refs[ntab], refs[ntab + 1], refs[ntab + 2]
        outs = refs[ntab + 3: ntab + 3 + n_out_total]
        rest = refs[ntab + 3 + n_out_total:]
        idx_s, idx_d = rest[0], rest[1]
        rows = rest[2:2 + RING]
        sems = rest[2 + RING:2 + 2 * RING]
        acc = rest[2 + 2 * RING]
        tbl = rest[2 + 2 * RING + 1]
        bulk = rest[2 + 2 * RING + 2]
        c = lax.axis_index("c")
        sid = lax.axis_index("s")

        def for_my_rows(fn):
            for k in range(KMAX):
                ci = sid + NS * k

                @pl.when(ci < NRC)
                def _():
                    fn(pl.multiple_of(ci * RCH, 8))

        def fire_zero():
            for_my_rows(lambda r: pltpu.async_copy(
                zrows, acc.at[pl.ds(r, RCH)], bulk))

        def drain_zero():
            for_my_rows(lambda r: pltpu.make_async_copy(
                zrows, acc.at[pl.ds(r, RCH)], bulk).wait())

        def fire_wb(out_ref):
            for_my_rows(lambda r: pltpu.async_copy(
                acc.at[pl.ds(r, RCH)], out_ref.at[pl.ds(r, RCH)], bulk))

        def drain_wb(out_ref):
            for_my_rows(lambda r: pltpu.make_async_copy(
                acc.at[pl.ds(r, RCH)], out_ref.at[pl.ds(r, RCH)], bulk).wait())

        def zero_sink():
            @pl.when(sid == 0)
            def _():
                pltpu.sync_copy(zrows.at[pl.ds(0, 16)], acc.at[pl.ds(N, 16)])

        fire_zero()
        zero_sink()
        drain_zero()
        plsc.subcore_barrier()

        def gather(tab, a, t):
            pltpu.async_copy(tbl.at[idx_s.at[a]], rows[t], sems[t])

        def wait_sem(tab, t):
            pltpu.make_async_copy(tbl.at[idx_s.at[0]], rows[t],
                                  sems[t]).wait()

        def scatter(a, t):
            pltpu.async_copy(rows[t], acc.at[idx_d.at[a]], sems[t], add=True)

        def run_half(tab, r0, nrows):
            pltpu.sync_copy(src_ref.at[pl.ds(r0, nrows)],
                            idx_s.at[pl.ds(0, nrows)])
            pltpu.sync_copy(dst_ref.at[pl.ds(r0, nrows)],
                            idx_d.at[pl.ds(0, nrows)])
            for t in range(RING):
                gather(tab, t, t)

            def pipe(g, _):
                a = RING * g
                for t in range(RING):
                    wait_sem(tab, t)
                    scatter(a + t, t)
                for t in range(RING):
                    wait_sem(tab, t)
                    gather(tab, a + RING + t, t)
                return 0

            lax.fori_loop(0, nrows // RING - 1, pipe, 0)
            a = nrows - RING
            for t in range(RING):
                wait_sem(tab, t)
                scatter(a + t, t)
            for t in range(RING):
                wait_sem(tab, t)

        def load_table(tab):
            for_my_rows(lambda r: pltpu.async_copy(
                tab.at[pl.ds(r, RCH)], tbl.at[pl.ds(r, RCH)], bulk))
            for_my_rows(lambda r: pltpu.make_async_copy(
                tab.at[pl.ds(r, RCH)], tbl.at[pl.ds(r, RCH)], bulk).wait())

        def run_block(b):
            tab = tabs[b]
            load_table(tab)
            plsc.subcore_barrier()
            if edge_split:
                base = c * (EROWS // NC) + sid * rpt
                run_half(tab, base, rpt)
            else:
                run_half(tab, sid * RPT, HPT)
                run_half(tab, sid * RPT + HPT, HPT)
            plsc.subcore_barrier()
            if edge_split:
                @pl.when(c == 0)
                def _():
                    fire_wb(outs[2 * b])
                    drain_wb(outs[2 * b])

                @pl.when(c == 1)
                def _():
                    fire_wb(outs[2 * b + 1])
                    drain_wb(outs[2 * b + 1])
            else:
                fire_wb(outs[b])
                drain_wb(outs[b])
            if b + NC < ntab or with_deg:
                fire_zero()
                zero_sink()
                drain_zero()
                plsc.subcore_barrier()

        for b in range(ntab):
            if edge_split:
                run_block(b)
            else:
                pl.when(c == b % NC)(functools.partial(run_block, b))

        if with_deg:
            one16 = jnp.ones((16,), jnp.float32)

            def oinit(i, _):
                for j in range(W // 16):
                    rows[0][i, pl.ds(16 * j, 16)] = one16
                return 0

            lax.fori_loop(0, CH, oinit, 0)
            drpt = RPT // NC
            r0 = c * (EROWS // NC) + sid * drpt
            pltpu.sync_copy(dst_ref.at[pl.ds(r0, drpt)],
                            idx_d.at[pl.ds(0, drpt)])
            for k in range(drpt):
                pltpu.async_copy(rows[0], acc.at[idx_d.at[k]], bulk, add=True)
            for k in range(drpt):
                pltpu.make_async_copy(rows[0], acc.at[idx_d.at[0]],
                                      bulk).wait()
            plsc.subcore_barrier()

            @pl.when(c == 0)
            def _():
                fire_wb(outs[nout])
                drain_wb(outs[nout])

            @pl.when(c == 1)
            def _():
                fire_wb(outs[nout + 1])
                drain_wb(outs[nout + 1])

    out_type = [jax.ShapeDtypeStruct((N, W), jnp.float32)] * n_out_total
    scratch = (
        [pltpu.VMEM((HPT, CH), jnp.int32)] * 2 +
        [pltpu.VMEM((CH, W), jnp.float32)] * RING +
        [pltpu.SemaphoreType.DMA] * RING +
        [pltpu.VMEM_SHARED((NP, W), jnp.float32),
         pltpu.VMEM_SHARED((NP, W), jnp.float32),
         pltpu.SemaphoreType.DMA]
    )
    return pl.kernel(body, out_type=tuple(out_type), mesh=mesh,
                     scratch_types=tuple(scratch),
                     compiler_params=pltpu.CompilerParams(
                         use_tc_tiling_on_sc=False))


BN = 400
GRID = N // BN


def _inv_deg(d0, d1):
    return 1.0 / jnp.maximum(d0[:, 0:1] + d1[:, 0:1], 1.0)


def _h1z1_body(*refs):
    x_ref = refs[0]
    a_refs = refs[1:5]
    d0_ref, d1_ref, w0_ref, w1_ref = refs[5], refs[6], refs[7], refs[8]
    o_refs = refs[9:17]
    inv = _inv_deg(d0_ref[...], d1_ref[...])
    agg = jnp.concatenate([a[...] for a in a_refs], axis=1)
    t = (x_ref[...] + agg * inv).astype(jnp.bfloat16)
    h1 = jnp.maximum(
        lax.dot_general(t, w0_ref[...], (((1,), (1,)), ((), ())),
                        preferred_element_type=jnp.float32),
        0.0).astype(jnp.bfloat16)
    z = lax.dot_general(h1, w1_ref[...], (((1,), (1,)), ((), ())),
                        preferred_element_type=jnp.float32)
    for j in range(8):
        o_refs[j][...] = z[:, W * j:W * (j + 1)]


def _h1z1(x, aggs, d0, d1, w0m, w1m):
    blk = lambda w: pl.BlockSpec((BN, w), lambda i: (i, 0))
    full = lambda shp: pl.BlockSpec(shp, lambda i: (0, 0))
    return pl.pallas_call(
        _h1z1_body,
        grid=(GRID,),
        in_specs=[blk(F)] + [blk(W)] * 4 + [blk(W), blk(W),
                  full((H, F)), full((H, H))],
        out_specs=[blk(W)] * 8,
        out_shape=[jax.ShapeDtypeStruct((N, W), jnp.float32)] * 8,
    )(x, *aggs, d0, d1, w0m, w1m)


def _h2z2_body(*refs):
    z_refs = refs[0:8]
    b_refs = refs[8:16]
    d0_ref, d1_ref, w2_ref, o_ref = refs[16], refs[17], refs[18], refs[19]
    inv = _inv_deg(d0_ref[...], d1_ref[...])
    acc = jnp.zeros((BN, W), jnp.float32)
    for b in range(8):
        h = jnp.maximum(z_refs[b][...] + b_refs[b][...] * inv,
                        0.0).astype(jnp.bfloat16)
        acc = acc + lax.dot_general(
            h, w2_ref[:, W * b:W * (b + 1)], (((1,), (1,)), ((), ())),
            preferred_element_type=jnp.float32)
    o_ref[...] = acc


def _h2z2(zs, bs, d0, d1, w2m):
    blk = lambda w: pl.BlockSpec((BN, w), lambda i: (i, 0))
    return pl.pallas_call(
        _h2z2_body,
        grid=(GRID,),
        in_specs=[blk(W)] * 16 + [blk(W), blk(W),
                  pl.BlockSpec((W, H), lambda i: (0, 0))],
        out_specs=blk(W),
        out_shape=jax.ShapeDtypeStruct((N, W), jnp.float32),
    )(*zs, *bs, d0, d1, w2m)


def _out_body(z2_ref, p0_ref, p1_ref, d0_ref, d1_ref, o_ref):
    inv = _inv_deg(d0_ref[...], d1_ref[...])
    t = z2_ref[...] + (p0_ref[...] + p1_ref[...]) * inv
    o_ref[...] = jnp.maximum(t, 0.0)[:, 0:C]


def _final(z2, p0, p1, d0, d1):
    blk = lambda w: pl.BlockSpec((BN, w), lambda i: (i, 0))
    return pl.pallas_call(
        _out_body,
        grid=(GRID,),
        in_specs=[blk(W)] * 5,
        out_specs=blk(C),
        out_shape=jax.ShapeDtypeStruct((N, C), jnp.float32),
    )(z2, p0, p1, d0, d1)


_sc_agg0 = _make_sc_agg(ntab=4, with_deg=True, edge_split=False)
_sc_agg1 = _make_sc_agg(ntab=8, with_deg=False, edge_split=False)
_sc_agg2 = _make_sc_agg(ntab=1, with_deg=False, edge_split=True)


def kernel(x, edge_index, snorm_n, snorm_e, adj_mask, W0, s0, W1, s1, W2, s2):
    src = edge_index[0]
    dst = edge_index[1]
    pad = E2 - E
    src2 = jnp.concatenate([src, jnp.zeros((pad,), jnp.int32)]
                           ).reshape(EROWS, CH)
    dst2 = jnp.concatenate([dst, jnp.full((pad,), SINK, jnp.int32)]
                           ).reshape(EROWS, CH)
    zrows = jnp.zeros((RCH, W), jnp.float32)

    w0m = _mask_weights(W0, s0, (H * F) // 2)
    w1m = _mask_weights(W1, s1, (H * H) // 2)
    w2p = jnp.pad(W2, ((0, W - C), (0, 0)))
    s2p = jnp.pad(s2, ((0, W - C), (0, 0)), constant_values=2.0)
    w2m = _mask_weights(w2p, s2p, (C * H) // 2)

    xblocks = [x[:, W * j:W * (j + 1)] for j in range(4)]
    *a_blocks, deg0, deg1 = _sc_agg0(*xblocks, src2, dst2, zrows)

    zs = _h1z1(x, a_blocks, deg0, deg1, w0m, w1m)
    bs = _sc_agg1(*zs, src2, dst2, zrows)
    z2 = _h2z2(zs, bs, deg0, deg1, w2m)
    p0, p1 = _sc_agg2(z2, src2, dst2, zrows)
    return _final(z2, p0, p1, deg0, deg1)


# speedup vs baseline: 1.0082x; 1.0082x over previous
import functools

import jax
import jax.numpy as jnp
from jax import lax
from jax.experimental import pallas as pl
from jax.experimental.pallas import tpu as pltpu
from jax.experimental.pallas import tpu_sc as plsc

N = 10000
E = 160000
F = 256
H = 512
C = 40
W = 64

NC = 2
NS = 16
CH = 128
RING = 4
EROWS = 1280
E2 = EROWS * CH
SINK = N
NP = N + 16
RPT = EROWS // NS
HPT = RPT // 2
NG = RPT // RING
RCH = 80
NRC = N // RCH
KMAX = (NRC + NS - 1) // NS


def _masks_body(ks, w0_ref, s0_ref, w1_ref, s1_ref, w2_ref, s2_ref,
                o0_ref, o1_ref, o2_ref):
    for k, w_ref, s_ref, o_ref in ((ks[0], w0_ref, s0_ref, o0_ref),
                                   (ks[1], w1_ref, s1_ref, o1_ref),
                                   (ks[2], w2_ref, s2_ref, o2_ref)):
        s = s_ref[...]
        bits = lax.bitcast_convert_type(s, jnp.int32)

        def step(_, lohi, bits=bits, k=k):
            lo, hi = lohi
            mid = (lo + hi) // 2
            cnt = jnp.sum((bits <= mid).astype(jnp.int32))
            take = cnt >= k + 1
            return jnp.where(take, lo, mid + 1), jnp.where(take, mid, hi)

        lo, _ = lax.fori_loop(0, 31, step,
                              (jnp.int32(0), jnp.int32(0x3F800000)))
        thr = lax.bitcast_convert_type(lo, jnp.float32)
        o_ref[...] = jnp.where(s < thr, 0.0, w_ref[...]).astype(jnp.bfloat16)


def _mask_weights3(w0, s0, w1, s1, w2, s2, ks):
    return pl.pallas_call(
        functools.partial(_masks_body, ks),
        out_shape=[jax.ShapeDtypeStruct(w0.shape, jnp.bfloat16),
                   jax.ShapeDtypeStruct(w1.shape, jnp.bfloat16),
                   jax.ShapeDtypeStruct(w2.shape, jnp.bfloat16)],
    )(w0, s0, w1, s1, w2, s2)


DEG_L = 8192
DEG_G = 163840 // DEG_L
NQ = 80


def _deg_body(dl_ref, ds_ref, o_ref):
    i = pl.program_id(0)
    d_lane = dl_ref[0]
    d_sub = ds_ref[...]
    qa = jnp.broadcast_to(d_lane // 128, (NQ, DEG_L))
    a2 = (qa == lax.broadcasted_iota(jnp.int32, (NQ, DEG_L), 0)
          ).astype(jnp.bfloat16)
    rb = jnp.broadcast_to(d_sub % 128, (DEG_L, 128))
    b2 = (rb == lax.broadcasted_iota(jnp.int32, (DEG_L, 128), 1)
          ).astype(jnp.bfloat16)
    contrib = lax.dot_general(a2, b2, (((1,), (0,)), ((), ())),
                              preferred_element_type=jnp.float32)

    @pl.when(i == 0)
    def _():
        o_ref[...] = jnp.zeros((NQ, 128), jnp.float32)

    o_ref[...] += contrib


def _degrees(dst_lanes, dst_subl):
    return pl.pallas_call(
        _deg_body,
        grid=(DEG_G,),
        in_specs=[pl.BlockSpec((1, 1, DEG_L), lambda i: (i, 0, 0)),
                  pl.BlockSpec((DEG_L, 1), lambda i: (i, 0))],
        out_specs=pl.BlockSpec((NQ, 128), lambda i: (0, 0)),
        out_shape=jax.ShapeDtypeStruct((NQ, 128), jnp.float32),
    )(dst_lanes, dst_subl)


def _make_sc_agg(ntab, edge_split):
    mesh = plsc.VectorSubcoreMesh(
        core_axis_name="c", subcore_axis_name="s", num_cores=NC,
        num_subcores=NS)
    nout = ntab * (NC if edge_split else 1)
    n_out_total = nout
    rpt = RPT if not edge_split else RPT // NC

    def body(*refs):
        tabs = refs[:ntab]
        src_ref, dst_ref, zrows = refs[ntab], refs[ntab + 1], refs[ntab + 2]
        outs = refs[ntab + 3: ntab + 3 + n_out_total]
        rest = refs[ntab + 3 + n_out_total:]
        idx_s, idx_d = rest[0], rest[1]
        rows = rest[2:2 + RING]
        sems = rest[2 + RING:2 + 2 * RING]
        acc = rest[2 + 2 * RING]
        tbl = rest[2 + 2 * RING + 1]
        bulk = rest[2 + 2 * RING + 2]
        c = lax.axis_index("c")
        sid = lax.axis_index("s")

        def for_my_rows(fn):
            for k in range(KMAX):
                ci = sid + NS * k

                @pl.when(ci < NRC)
                def _():
                    fn(pl.multiple_of(ci * RCH, 8))

        def fire_zero():
            for_my_rows(lambda r: pltpu.async_copy(
                zrows, acc.at[pl.ds(r, RCH)], bulk))

        def drain_zero():
            for_my_rows(lambda r: pltpu.make_async_copy(
                zrows, acc.at[pl.ds(r, RCH)], bulk).wait())

        def fire_wb(out_ref):
            for_my_rows(lambda r: pltpu.async_copy(
                acc.at[pl.ds(r, RCH)], out_ref.at[pl.ds(r, RCH)], bulk))

        def drain_wb(out_ref):
            for_my_rows(lambda r: pltpu.make_async_copy(
                acc.at[pl.ds(r, RCH)], out_ref.at[pl.ds(r, RCH)], bulk).wait())

        def zero_sink():
            @pl.when(sid == 0)
            def _():
                pltpu.sync_copy(zrows.at[pl.ds(0, 16)], acc.at[pl.ds(N, 16)])

        fire_zero()
        zero_sink()
        drain_zero()
        plsc.subcore_barrier()

        def gather(tab, a, t):
            pltpu.async_copy(tbl.at[idx_s.at[a]], rows[t], sems[t])

        def wait_sem(tab, t):
            pltpu.make_async_copy(tbl.at[idx_s.at[0]], rows[t],
                                  sems[t]).wait()

        def scatter(a, t):
            pltpu.async_copy(rows[t], acc.at[idx_d.at[a]], sems[t], add=True)

        def run_half(tab, r0, nrows):
            pltpu.sync_copy(src_ref.at[pl.ds(r0, nrows)],
                            idx_s.at[pl.ds(0, nrows)])
            pltpu.sync_copy(dst_ref.at[pl.ds(r0, nrows)],
                            idx_d.at[pl.ds(0, nrows)])
            for t in range(RING):
                gather(tab, t, t)

            def pipe(g, _):
                a = RING * g
                for t in range(RING):
                    wait_sem(tab, t)
                    scatter(a + t, t)
                for t in range(RING):
                    wait_sem(tab, t)
                    gather(tab, a + RING + t, t)
                return 0

            lax.fori_loop(0, nrows // RING - 1, pipe, 0)
            a = nrows - RING
            for t in range(RING):
                wait_sem(tab, t)
                scatter(a + t, t)
            for t in range(RING):
                wait_sem(tab, t)

        def load_table(tab):
            for_my_rows(lambda r: pltpu.async_copy(
                tab.at[pl.ds(r, RCH)], tbl.at[pl.ds(r, RCH)], bulk))
            for_my_rows(lambda r: pltpu.make_async_copy(
                tab.at[pl.ds(r, RCH)], tbl.at[pl.ds(r, RCH)], bulk).wait())

        def run_block(b):
            tab = tabs[b]
            load_table(tab)
            plsc.subcore_barrier()
            if edge_split:
                base = c * (EROWS // NC) + sid * rpt
                run_half(tab, base, rpt)
            else:
                run_half(tab, sid * RPT, HPT)
                run_half(tab, sid * RPT + HPT, HPT)
            plsc.subcore_barrier()
            if edge_split:
                @pl.when(c == 0)
                def _():
                    fire_wb(outs[2 * b])
                    drain_wb(outs[2 * b])

                @pl.when(c == 1)
                def _():
                    fire_wb(outs[2 * b + 1])
                    drain_wb(outs[2 * b + 1])
            else:
                fire_wb(outs[b])
                drain_wb(outs[b])
            if b + NC < ntab:
                fire_zero()
                zero_sink()
                drain_zero()
                plsc.subcore_barrier()

        for b in range(ntab):
            if edge_split:
                run_block(b)
            else:
                pl.when(c == b % NC)(functools.partial(run_block, b))

    out_type = [jax.ShapeDtypeStruct((N, W), jnp.float32)] * n_out_total
    scratch = (
        [pltpu.VMEM((HPT, CH), jnp.int32)] * 2 +
        [pltpu.VMEM((CH, W), jnp.float32)] * RING +
        [pltpu.SemaphoreType.DMA] * RING +
        [pltpu.VMEM_SHARED((NP, W), jnp.float32),
         pltpu.VMEM_SHARED((NP, W), jnp.float32),
         pltpu.SemaphoreType.DMA]
    )
    return pl.kernel(body, out_type=tuple(out_type), mesh=mesh,
                     scratch_types=tuple(scratch),
                     compiler_params=pltpu.CompilerParams(
                         use_tc_tiling_on_sc=False))


BN = 400
GRID = N // BN


def _inv_deg(d):
    return 1.0 / jnp.maximum(d, 1.0)


def _h1z1_body(*refs):
    x_ref = refs[0]
    a_refs = refs[1:5]
    deg_ref, w0_ref, w1_ref = refs[5], refs[6], refs[7]
    o_refs = refs[8:16]
    inv = _inv_deg(deg_ref[...])
    agg = jnp.concatenate([a[...] for a in a_refs], axis=1)
    t = (x_ref[...] + agg * inv).astype(jnp.bfloat16)
    h1 = jnp.maximum(
        lax.dot_general(t, w0_ref[...], (((1,), (1,)), ((), ())),
                        preferred_element_type=jnp.float32),
        0.0).astype(jnp.bfloat16)
    z = lax.dot_general(h1, w1_ref[...], (((1,), (1,)), ((), ())),
                        preferred_element_type=jnp.float32)
    for j in range(8):
        o_refs[j][...] = z[:, W * j:W * (j + 1)]


def _h1z1(x, aggs, deg, w0m, w1m):
    blk = lambda w: pl.BlockSpec((BN, w), lambda i: (i, 0))
    full = lambda shp: pl.BlockSpec(shp, lambda i: (0, 0))
    return pl.pallas_call(
        _h1z1_body,
        grid=(GRID,),
        in_specs=[blk(F)] + [blk(W)] * 4 + [blk(1),
                  full((H, F)), full((H, H))],
        out_specs=[blk(W)] * 8,
        out_shape=[jax.ShapeDtypeStruct((N, W), jnp.float32)] * 8,
    )(x, *aggs, deg, w0m, w1m)


def _h2z2_body(*refs):
    z_refs = refs[0:8]
    b_refs = refs[8:16]
    deg_ref, w2_ref, o_ref = refs[16], refs[17], refs[18]
    inv = _inv_deg(deg_ref[...])
    acc = jnp.zeros((BN, W), jnp.float32)
    for b in range(8):
        h = jnp.maximum(z_refs[b][...] + b_refs[b][...] * inv,
                        0.0).astype(jnp.bfloat16)
        acc = acc + lax.dot_general(
            h, w2_ref[:, W * b:W * (b + 1)], (((1,), (1,)), ((), ())),
            preferred_element_type=jnp.float32)
    o_ref[...] = acc


def _h2z2(zs, bs, deg, w2m):
    blk = lambda w: pl.BlockSpec((BN, w), lambda i: (i, 0))
    return pl.pallas_call(
        _h2z2_body,
        grid=(GRID,),
        in_specs=[blk(W)] * 16 + [blk(1),
                  pl.BlockSpec((W, H), lambda i: (0, 0))],
        out_specs=blk(W),
        out_shape=jax.ShapeDtypeStruct((N, W), jnp.float32),
    )(*zs, *bs, deg, w2m)


def _out_body(z2_ref, p0_ref, p1_ref, deg_ref, o_ref):
    inv = _inv_deg(deg_ref[...])
    t = z2_ref[...] + (p0_ref[...] + p1_ref[...]) * inv
    o_ref[...] = jnp.maximum(t, 0.0)[:, 0:C]


def _final(z2, p0, p1, deg):
    blk = lambda w: pl.BlockSpec((BN, w), lambda i: (i, 0))
    return pl.pallas_call(
        _out_body,
        grid=(GRID,),
        in_specs=[blk(W)] * 3 + [blk(1)],
        out_specs=blk(C),
        out_shape=jax.ShapeDtypeStruct((N, C), jnp.float32),
    )(z2, p0, p1, deg)


_sc_agg0 = _make_sc_agg(ntab=4, edge_split=False)
_sc_agg1 = _make_sc_agg(ntab=8, edge_split=False)
_sc_agg2 = _make_sc_agg(ntab=1, edge_split=True)


def kernel(x, edge_index, snorm_n, snorm_e, adj_mask, W0, s0, W1, s1, W2, s2):
    src = edge_index[0]
    dst = edge_index[1]
    pad = E2 - E
    src2 = jnp.concatenate([src, jnp.zeros((pad,), jnp.int32)]
                           ).reshape(EROWS, CH)
    dst2 = jnp.concatenate([dst, jnp.full((pad,), SINK, jnp.int32)]
                           ).reshape(EROWS, CH)
    zrows = jnp.zeros((RCH, W), jnp.float32)

    w2p = jnp.pad(W2, ((0, W - C), (0, 0)))
    s2p = jnp.pad(s2, ((0, W - C), (0, 0)), constant_values=2.0)
    w0m, w1m, w2m = _mask_weights3(W0, s0, W1, s1, w2p, s2p,
                                   ((H * F) // 2, (H * H) // 2, (C * H) // 2))

    dflat = dst2.reshape(E2)
    deg = _degrees(dflat.reshape(DEG_G, 1, DEG_L),
                   dflat.reshape(E2, 1)).reshape(NQ * 128)[:N].reshape(N, 1)

    xblocks = [x[:, W * j:W * (j + 1)] for j in range(4)]
    a_blocks = _sc_agg0(*xblocks, src2, dst2, zrows)

    zs = _h1z1(x, a_blocks, deg, w0m, w1m)
    bs = _sc_agg1(*zs, src2, dst2, zrows)
    z2 = _h2z2(zs, bs, deg, w2m)
    p0, p1 = _sc_agg2(z2, src2, dst2, zrows)
    return _final(z2, p0, p1, deg)


# speedup vs baseline: 1.0246x; 1.0163x over previous
import functools

import jax
import jax.numpy as jnp
from jax import lax
from jax.experimental import pallas as pl
from jax.experimental.pallas import tpu as pltpu
from jax.experimental.pallas import tpu_sc as plsc

N = 10000
E = 160000
F = 256
H = 512
C = 40
W = 64

NC = 2
NS = 16
CH = 128
RING = 4
EROWS = 1280
E2 = EROWS * CH
SINK = N
NP = N + 16
RPT = EROWS // NS
HPT = RPT // 2
NG = RPT // RING
RCH = 80
NRC = N // RCH
KMAX = (NRC + NS - 1) // NS


def _masks_body(ks, w0_ref, s0_ref, w1_ref, s1_ref, w2_ref, s2_ref,
                o0_ref, o1_ref, o2_ref):
    for k, w_ref, s_ref, o_ref in ((ks[0], w0_ref, s0_ref, o0_ref),
                                   (ks[1], w1_ref, s1_ref, o1_ref),
                                   (ks[2], w2_ref, s2_ref, o2_ref)):
        s = s_ref[...]
        bits = lax.bitcast_convert_type(s, jnp.int32)

        def step(_, lohi, bits=bits, k=k):
            lo, hi = lohi
            mid = (lo + hi) // 2
            cnt = jnp.sum((bits <= mid).astype(jnp.int32))
            take = cnt >= k + 1
            return jnp.where(take, lo, mid + 1), jnp.where(take, mid, hi)

        lo, _ = lax.fori_loop(0, 31, step,
                              (jnp.int32(0), jnp.int32(0x3F800000)))
        thr = lax.bitcast_convert_type(lo, jnp.float32)
        o_ref[...] = jnp.where(s < thr, 0.0, w_ref[...]).astype(jnp.bfloat16)


def _mask_weights3(w0, s0, w1, s1, w2, s2, ks):
    return pl.pallas_call(
        functools.partial(_masks_body, ks),
        out_shape=[jax.ShapeDtypeStruct(w0.shape, jnp.bfloat16),
                   jax.ShapeDtypeStruct(w1.shape, jnp.bfloat16),
                   jax.ShapeDtypeStruct(w2.shape, jnp.bfloat16)],
    )(w0, s0, w1, s1, w2, s2)


DEG_L = 8192
DEG_G = 163840 // DEG_L
NQ = 80


def _deg_body(dl_ref, ds_ref, o_ref):
    i = pl.program_id(0)
    d_lane = dl_ref[0]
    d_sub = ds_ref[...]
    qa = jnp.broadcast_to(d_lane // 128, (NQ, DEG_L))
    a2 = (qa == lax.broadcasted_iota(jnp.int32, (NQ, DEG_L), 0)
          ).astype(jnp.bfloat16)
    rb = jnp.broadcast_to(d_sub % 128, (DEG_L, 128))
    b2 = (rb == lax.broadcasted_iota(jnp.int32, (DEG_L, 128), 1)
          ).astype(jnp.bfloat16)
    contrib = lax.dot_general(a2, b2, (((1,), (0,)), ((), ())),
                              preferred_element_type=jnp.float32)

    @pl.when(i == 0)
    def _():
        o_ref[...] = jnp.zeros((NQ, 128), jnp.float32)

    o_ref[...] += contrib


def _degrees(dst_lanes, dst_subl):
    return pl.pallas_call(
        _deg_body,
        grid=(DEG_G,),
        in_specs=[pl.BlockSpec((1, 1, DEG_L), lambda i: (i, 0, 0)),
                  pl.BlockSpec((DEG_L, 1), lambda i: (i, 0))],
        out_specs=pl.BlockSpec((NQ, 128), lambda i: (0, 0)),
        out_shape=jax.ShapeDtypeStruct((NQ, 128), jnp.float32),
    )(dst_lanes, dst_subl)


def _make_sc_agg(ntab, edge_split):
    mesh = plsc.VectorSubcoreMesh(
        core_axis_name="c", subcore_axis_name="s", num_cores=NC,
        num_subcores=NS)
    nout = ntab * (NC if edge_split else 1)
    n_out_total = nout
    rpt = RPT if not edge_split else RPT // NC

    def body(*refs):
        tabs = refs[:ntab]
        src_ref, dst_ref, zrows = refs[ntab], refs[ntab + 1], refs[ntab + 2]
        outs = refs[ntab + 3: ntab + 3 + n_out_total]
        rest = refs[ntab + 3 + n_out_total:]
        idx_s, idx_d = rest[0], rest[1]
        rows = rest[2:2 + RING]
        sems = rest[2 + RING:2 + 2 * RING]
        acc = rest[2 + 2 * RING]
        tbl = rest[2 + 2 * RING + 1]
        bulk = rest[2 + 2 * RING + 2]
        c = lax.axis_index("c")
        sid = lax.axis_index("s")

        def for_my_rows(fn):
            for k in range(KMAX):
                ci = sid + NS * k

                @pl.when(ci < NRC)
                def _():
                    fn(pl.multiple_of(ci * RCH, 8))

        def fire_zero():
            for_my_rows(lambda r: pltpu.async_copy(
                zrows, acc.at[pl.ds(r, RCH)], bulk))

        def drain_zero():
            for_my_rows(lambda r: pltpu.make_async_copy(
                zrows, acc.at[pl.ds(r, RCH)], bulk).wait())

        def fire_wb(out_ref):
            for_my_rows(lambda r: pltpu.async_copy(
                acc.at[pl.ds(r, RCH)], out_ref.at[pl.ds(r, RCH)], bulk))

        def drain_wb(out_ref):
            for_my_rows(lambda r: pltpu.make_async_copy(
                acc.at[pl.ds(r, RCH)], out_ref.at[pl.ds(r, RCH)], bulk).wait())

        def zero_sink():
            @pl.when(sid == 0)
            def _():
                pltpu.sync_copy(zrows.at[pl.ds(0, 16)], acc.at[pl.ds(N, 16)])

        fire_zero()
        zero_sink()
        drain_zero()
        plsc.subcore_barrier()

        def gather(tab, a, t):
            pltpu.async_copy(tbl.at[idx_s.at[a]], rows[t], sems[t])

        def wait_sem(tab, t):
            pltpu.make_async_copy(tbl.at[idx_s.at[0]], rows[t],
                                  sems[t]).wait()

        def scatter(a, t):
            pltpu.async_copy(rows[t], acc.at[idx_d.at[a]], sems[t], add=True)

        def run_half(tab, r0, nrows):
            pltpu.sync_copy(src_ref.at[pl.ds(r0, nrows)],
                            idx_s.at[pl.ds(0, nrows)])
            pltpu.sync_copy(dst_ref.at[pl.ds(r0, nrows)],
                            idx_d.at[pl.ds(0, nrows)])
            for t in range(RING):
                gather(tab, t, t)

            def pipe(g, _):
                a = RING * g
                for t in range(RING):
                    wait_sem(tab, t)
                    scatter(a + t, t)
                for t in range(RING):
                    wait_sem(tab, t)
                    gather(tab, a + RING + t, t)
                return 0

            lax.fori_loop(0, nrows // RING - 1, pipe, 0)
            a = nrows - RING
            for t in range(RING):
                wait_sem(tab, t)
                scatter(a + t, t)
            for t in range(RING):
                wait_sem(tab, t)

        def load_table(tab):
            for_my_rows(lambda r: pltpu.async_copy(
                tab.at[pl.ds(r, RCH)], tbl.at[pl.ds(r, RCH)], bulk))
            for_my_rows(lambda r: pltpu.make_async_copy(
                tab.at[pl.ds(r, RCH)], tbl.at[pl.ds(r, RCH)], bulk).wait())

        def run_block(b):
            tab = tabs[b]
            load_table(tab)
            plsc.subcore_barrier()
            if edge_split:
                base = c * (EROWS // NC) + sid * rpt
                run_half(tab, base, rpt)
            else:
                run_half(tab, sid * RPT, HPT)
                run_half(tab, sid * RPT + HPT, HPT)
            plsc.subcore_barrier()
            if edge_split:
                @pl.when(c == 0)
                def _():
                    fire_wb(outs[2 * b])
                    drain_wb(outs[2 * b])

                @pl.when(c == 1)
                def _():
                    fire_wb(outs[2 * b + 1])
                    drain_wb(outs[2 * b + 1])
            else:
                fire_wb(outs[b])
                drain_wb(outs[b])
            if b + NC < ntab:
                fire_zero()
                zero_sink()
                drain_zero()
                plsc.subcore_barrier()

        for b in range(ntab):
            if edge_split:
                run_block(b)
            else:
                pl.when(c == b % NC)(functools.partial(run_block, b))

    out_type = [jax.ShapeDtypeStruct((N, W), jnp.float32)] * n_out_total
    scratch = (
        [pltpu.VMEM((HPT, CH), jnp.int32)] * 2 +
        [pltpu.VMEM((CH, W), jnp.float32)] * RING +
        [pltpu.SemaphoreType.DMA] * RING +
        [pltpu.VMEM_SHARED((NP, W), jnp.float32),
         pltpu.VMEM_SHARED((NP, W), jnp.float32),
         pltpu.SemaphoreType.DMA]
    )
    return pl.kernel(body, out_type=tuple(out_type), mesh=mesh,
                     scratch_types=tuple(scratch),
                     compiler_params=pltpu.CompilerParams(
                         use_tc_tiling_on_sc=False))


BN = 400
GRID = N // BN


def _inv_deg(d):
    return 1.0 / jnp.maximum(d, 1.0)


def _h1z1_body(*refs):
    x_ref = refs[0]
    a_refs = refs[1:5]
    deg_ref, w0_ref, w1_ref = refs[5], refs[6], refs[7]
    o_refs = refs[8:16]
    inv = _inv_deg(deg_ref[...])
    agg = jnp.concatenate([a[...] for a in a_refs], axis=1)
    t = (x_ref[...] + agg * inv).astype(jnp.bfloat16)
    h1 = jnp.maximum(
        lax.dot_general(t, w0_ref[...], (((1,), (1,)), ((), ())),
                        preferred_element_type=jnp.float32),
        0.0).astype(jnp.bfloat16)
    z = lax.dot_general(h1, w1_ref[...], (((1,), (1,)), ((), ())),
                        preferred_element_type=jnp.float32)
    for j in range(8):
        o_refs[j][...] = z[:, W * j:W * (j + 1)]


def _h1z1(x, aggs, deg, w0m, w1m):
    blk = lambda w: pl.BlockSpec((BN, w), lambda i: (i, 0))
    full = lambda shp: pl.BlockSpec(shp, lambda i: (0, 0))
    return pl.pallas_call(
        _h1z1_body,
        grid=(GRID,),
        in_specs=[blk(F)] + [blk(W)] * 4 + [blk(1),
                  full((H, F)), full((H, H))],
        out_specs=[blk(W)] * 8,
        out_shape=[jax.ShapeDtypeStruct((N, W), jnp.float32)] * 8,
    )(x, *aggs, deg, w0m, w1m)


def _h2z2_body(*refs):
    z_refs = refs[0:8]
    b_refs = refs[8:16]
    deg_ref, w2_ref, o_ref = refs[16], refs[17], refs[18]
    inv = _inv_deg(deg_ref[...])
    acc = jnp.zeros((BN, W), jnp.float32)
    for b in range(8):
        h = jnp.maximum(z_refs[b][...] + b_refs[b][...] * inv,
                        0.0).astype(jnp.bfloat16)
        acc = acc + lax.dot_general(
            h, w2_ref[:, W * b:W * (b + 1)], (((1,), (1,)), ((), ())),
            preferred_element_type=jnp.float32)
    o_ref[...] = acc


def _h2z2(zs, bs, deg, w2m):
    blk = lambda w: pl.BlockSpec((BN, w), lambda i: (i, 0))
    return pl.pallas_call(
        _h2z2_body,
        grid=(GRID,),
        in_specs=[blk(W)] * 16 + [blk(1),
                  pl.BlockSpec((W, H), lambda i: (0, 0))],
        out_specs=blk(W),
        out_shape=jax.ShapeDtypeStruct((N, W), jnp.float32),
    )(*zs, *bs, deg, w2m)


def _out_body(z2_ref, p0_ref, p1_ref, deg_ref, o_ref):
    inv = _inv_deg(deg_ref[...])
    t = z2_ref[...] + (p0_ref[...] + p1_ref[...]) * inv
    o_ref[...] = jnp.maximum(t, 0.0)[:, 0:C]


def _final(z2, p0, p1, deg):
    blk = lambda w: pl.BlockSpec((BN, w), lambda i: (i, 0))
    return pl.pallas_call(
        _out_body,
        grid=(GRID,),
        in_specs=[blk(W)] * 3 + [blk(1)],
        out_specs=blk(C),
        out_shape=jax.ShapeDtypeStruct((N, C), jnp.float32),
    )(z2, p0, p1, deg)


_sc_agg0 = _make_sc_agg(ntab=4, edge_split=False)
_sc_agg1 = _make_sc_agg(ntab=8, edge_split=False)
_sc_agg2 = _make_sc_agg(ntab=1, edge_split=True)


def kernel(x, edge_index, snorm_n, snorm_e, adj_mask, W0, s0, W1, s1, W2, s2):
    src = edge_index[0]
    dst = edge_index[1]
    pad = E2 - E
    src2 = jnp.concatenate([src, jnp.zeros((pad,), jnp.int32)]
                           ).reshape(EROWS, CH)
    dst2 = jnp.concatenate([dst, jnp.full((pad,), SINK, jnp.int32)]
                           ).reshape(EROWS, CH)
    zrows = jnp.zeros((RCH, W), jnp.float32)

    xblocks = [x[:, W * j:W * (j + 1)] for j in range(4)]
    a_blocks = _sc_agg0(*xblocks, src2, dst2, zrows)

    w2p = jnp.pad(W2, ((0, W - C), (0, 0)))
    s2p = jnp.pad(s2, ((0, W - C), (0, 0)), constant_values=2.0)
    w0m, w1m, w2m = _mask_weights3(W0, s0, W1, s1, w2p, s2p,
                                   ((H * F) // 2, (H * H) // 2, (C * H) // 2))

    dflat = dst2.reshape(E2)
    deg = _degrees(dflat.reshape(DEG_G, 1, DEG_L),
                   dflat.reshape(E2, 1)).reshape(NQ * 128)[:N].reshape(N, 1)

    zs = _h1z1(x, a_blocks, deg, w0m, w1m)
    bs = _sc_agg1(*zs, src2, dst2, zrows)
    z2 = _h2z2(zs, bs, deg, w2m)
    p0, p1 = _sc_agg2(z2, src2, dst2, zrows)
    return _final(z2, p0, p1, deg)
